# baseline (speedup 1.0000x reference)
import jax
import jax.numpy as jnp
from jax.experimental import pallas as pl

N = 207
B = 16
L = 8
HORIZON = 8
U = 64
NUM_MAT = 5
DIN = 80
F32 = jnp.float32
BF16 = jnp.bfloat16

_DNT = (((0,), (0,)), ((), ()))


def _fwd_kernel(x_all_ref, s0t_ref, s1t_ref,
                w0_ru_e_ref, w4_ru_e_ref, b_ru_e_ref,
                w0_c_e_ref, w4_c_e_ref, b_c_e_ref,
                w0_ru_d_ref, w4_ru_d_ref, b_ru_d_ref,
                w0_c_d_ref, w4_c_d_ref, b_c_d_ref,
                w_projt_ref, b_proj_ref,
                out_ref):
    s0t = s0t_ref[...]
    s1t = s1t_ref[...]

    def dotT(a, w):
        return jax.lax.dot_general(a, w, _DNT, preferred_element_type=F32)

    def gconv(cat, w0, w4, b):
        g0 = jnp.dot(cat, w0, preferred_element_type=F32)
        cat3 = cat.reshape(B, N, DIN)
        z1a = jnp.concatenate(
            [dotT(cat3[i], s0t).astype(BF16) for i in range(B)], axis=0)
        z1b = jnp.concatenate(
            [dotT(cat3[i], s1t).astype(BF16) for i in range(B)], axis=0)
        z2a = jnp.dot(z1a, s0t, preferred_element_type=F32).astype(BF16)
        z2b = jnp.dot(z1b, s1t, preferred_element_type=F32).astype(BF16)
        gates = []
        for i in range(B):
            sl = slice(i * DIN, (i + 1) * DIN)
            gates.append(dotT(z1a[sl], w4[0]) + dotT(z2a[sl], w4[1])
                         + dotT(z1b[sl], w4[2]) + dotT(z2b[sl], w4[3]))
        return g0 + jnp.concatenate(gates, axis=0) + b

    def cell(x16, h, w0_ru, w4_ru, b_ru, w0_c, w4_c, b_c):
        cat = jnp.concatenate([h.astype(BF16), x16], axis=1)
        ru = jax.nn.sigmoid(gconv(cat, w0_ru, w4_ru, b_ru))
        r = ru[:, :U]
        u = ru[:, U:]
        cat2 = jnp.concatenate([(r * h).astype(BF16), x16], axis=1)
        c = jnp.tanh(gconv(cat2, w0_c, w4_c, b_c))
        return u * h + (1.0 - u) * c

    w0_ru_e = w0_ru_e_ref[...]
    w4_ru_e = w4_ru_e_ref[...]
    b_ru_e = b_ru_e_ref[...]
    w0_c_e = w0_c_e_ref[...]
    w4_c_e = w4_c_e_ref[...]
    b_c_e = b_c_e_ref[...]

    def enc_body(t, h):
        x16 = x_all_ref[pl.ds(t, 1)].reshape(B * N, DIN - U)
        return cell(x16, h, w0_ru_e, w4_ru_e, b_ru_e, w0_c_e, w4_c_e, b_c_e)

    h = jax.lax.fori_loop(0, L, enc_body, jnp.zeros((B * N, U), F32))

    w0_ru_d = w0_ru_d_ref[...]
    w4_ru_d = w4_ru_d_ref[...]
    b_ru_d = b_ru_d_ref[...]
    w0_c_d = w0_c_d_ref[...]
    w4_c_d = w4_c_d_ref[...]
    b_c_d = b_c_d_ref[...]
    w_projt = w_projt_ref[...]
    b_proj = b_proj_ref[0, 0]

    def dec_body(t, h):
        projt = jnp.dot(w_projt, h.T, preferred_element_type=F32) + b_proj
        xin = jnp.where(t == 0, jnp.zeros_like(projt), projt).T
        x16 = jnp.pad(xin, ((0, 0), (0, DIN - U - 1))).astype(BF16)
        h2 = cell(x16, h, w0_ru_d, w4_ru_d, b_ru_d, w0_c_d, w4_c_d, b_c_d)
        proj2 = jnp.dot(w_projt, h2.T, preferred_element_type=F32) + b_proj
        out_ref[pl.ds(t, 1)] = proj2
        return h2

    jax.lax.fori_loop(0, HORIZON, dec_body, h)


def _prep_w(w, dx, dout):
    din = dx + U
    wm = w.reshape(din, NUM_MAT, dout)
    wm = jnp.concatenate(
        [wm[dx:], wm[:dx], jnp.zeros((DIN - din, NUM_MAT, dout), F32)],
        axis=0)
    w0 = wm[:, 0] - wm[:, 2] - wm[:, 4]
    w4 = jnp.stack(
        [wm[:, 1], 2.0 * wm[:, 2], wm[:, 3], 2.0 * wm[:, 4]], axis=0)
    return w0.astype(BF16), w4.astype(BF16)


def kernel(inputs, support0, support1, W_ru_e, b_ru_e, W_c_e, b_c_e,
           W_ru_d, b_ru_d, W_c_d, b_c_d, W_proj, b_proj):
    x_all = jnp.pad(inputs.reshape(L, B * N, L),
                    ((0, 0), (0, 0), (0, DIN - U - L))).astype(BF16)

    w0_ru_e, w4_ru_e = _prep_w(W_ru_e, L, 2 * U)
    w0_c_e, w4_c_e = _prep_w(W_c_e, L, U)
    w0_ru_d, w4_ru_d = _prep_w(W_ru_d, 1, 2 * U)
    w0_c_d, w4_c_d = _prep_w(W_c_d, 1, U)

    out = pl.pallas_call(
        _fwd_kernel,
        out_shape=jax.ShapeDtypeStruct((HORIZON, B * N), F32),
    )(x_all, support0.T.astype(BF16), support1.T.astype(BF16),
      w0_ru_e, w4_ru_e, b_ru_e.reshape(1, 2 * U),
      w0_c_e, w4_c_e, b_c_e.reshape(1, U),
      w0_ru_d, w4_ru_d, b_ru_d.reshape(1, 2 * U),
      w0_c_d, w4_c_d, b_c_d.reshape(1, U),
      W_proj.T, b_proj.reshape(1, 1))

    return out.reshape(HORIZON, B, N)


# speedup vs baseline: 1.1494x; 1.1494x over previous
import jax
import jax.numpy as jnp
from jax.experimental import pallas as pl

N = 207
B = 16
L = 8
HORIZON = 8
U = 64
NUM_MAT = 5
DIN = 80
F32 = jnp.float32
BF16 = jnp.bfloat16

_DNT = (((0,), (0,)), ((), ()))


def _fwd_kernel(x_all_ref, s0t_ref, s1t_ref,
                w0_ru_e_ref, w4_ru_e_ref, b_ru_e_ref,
                w0_c_e_ref, w4_c_e_ref, b_c_e_ref,
                w0_ru_d_ref, w4_ru_d_ref, b_ru_d_ref,
                w0_c_d_ref, w4_c_d_ref, b_c_d_ref,
                w_projt_ref, b_proj_ref,
                out_ref):
    s0t = s0t_ref[...]
    s1t = s1t_ref[...]

    def dotT(a, w):
        return jax.lax.dot_general(a, w, _DNT, preferred_element_type=F32)

    def gconv(cat, w0, w4, b):
        g0 = jnp.dot(cat, w0, preferred_element_type=F32)
        cat3 = cat.reshape(B, N, DIN)
        z1a = jnp.concatenate(
            [dotT(cat3[i], s0t).astype(BF16) for i in range(B)], axis=0)
        z1b = jnp.concatenate(
            [dotT(cat3[i], s1t).astype(BF16) for i in range(B)], axis=0)
        z2a = jnp.dot(z1a, s0t, preferred_element_type=F32).astype(BF16)
        z2b = jnp.dot(z1b, s1t, preferred_element_type=F32).astype(BF16)
        gates = []
        for i in range(B):
            sl = slice(i * DIN, (i + 1) * DIN)
            zcat = jnp.concatenate([z1a[sl], z2a[sl], z1b[sl], z2b[sl]], axis=0)
            gates.append(dotT(zcat, w4))
        return g0 + jnp.concatenate(gates, axis=0) + b

    def cell(x16, h, w0_ru, w4_ru, b_ru, w0_c, w4_c, b_c):
        cat = jnp.concatenate([h.astype(BF16), x16], axis=1)
        ru = jax.nn.sigmoid(gconv(cat, w0_ru, w4_ru, b_ru))
        r = ru[:, :U]
        u = ru[:, U:]
        cat2 = jnp.concatenate([(r * h).astype(BF16), x16], axis=1)
        c = jnp.tanh(gconv(cat2, w0_c, w4_c, b_c))
        return u * h + (1.0 - u) * c

    w0_ru_e = w0_ru_e_ref[...]
    w4_ru_e = w4_ru_e_ref[...]
    b_ru_e = b_ru_e_ref[...]
    w0_c_e = w0_c_e_ref[...]
    w4_c_e = w4_c_e_ref[...]
    b_c_e = b_c_e_ref[...]

    def enc_body(t, h):
        x16 = x_all_ref[pl.ds(t, 1)].reshape(B * N, DIN - U)
        return cell(x16, h, w0_ru_e, w4_ru_e, b_ru_e, w0_c_e, w4_c_e, b_c_e)

    h = jax.lax.fori_loop(0, L, enc_body, jnp.zeros((B * N, U), F32))

    w0_ru_d = w0_ru_d_ref[...]
    w4_ru_d = w4_ru_d_ref[...]
    b_ru_d = b_ru_d_ref[...]
    w0_c_d = w0_c_d_ref[...]
    w4_c_d = w4_c_d_ref[...]
    b_c_d = b_c_d_ref[...]
    w_projt = w_projt_ref[...]
    b_proj = b_proj_ref[0, 0]

    def dec_body(t, h):
        projt = jnp.dot(w_projt, h.T, preferred_element_type=F32) + b_proj
        xin = jnp.where(t == 0, jnp.zeros_like(projt), projt).T
        x16 = jnp.pad(xin, ((0, 0), (0, DIN - U - 1))).astype(BF16)
        h2 = cell(x16, h, w0_ru_d, w4_ru_d, b_ru_d, w0_c_d, w4_c_d, b_c_d)
        proj2 = jnp.dot(w_projt, h2.T, preferred_element_type=F32) + b_proj
        out_ref[pl.ds(t, 1)] = proj2
        return h2

    jax.lax.fori_loop(0, HORIZON, dec_body, h)


def _prep_w(w, dx, dout):
    din = dx + U
    wm = w.reshape(din, NUM_MAT, dout)
    wm = jnp.concatenate(
        [wm[dx:], wm[:dx], jnp.zeros((DIN - din, NUM_MAT, dout), F32)],
        axis=0)
    w0 = wm[:, 0] - wm[:, 2] - wm[:, 4]
    w4 = jnp.concatenate(
        [wm[:, 1], 2.0 * wm[:, 2], wm[:, 3], 2.0 * wm[:, 4]], axis=0)
    return w0.astype(BF16), w4.astype(BF16)


def kernel(inputs, support0, support1, W_ru_e, b_ru_e, W_c_e, b_c_e,
           W_ru_d, b_ru_d, W_c_d, b_c_d, W_proj, b_proj):
    x_all = jnp.pad(inputs.reshape(L, B * N, L),
                    ((0, 0), (0, 0), (0, DIN - U - L))).astype(BF16)

    w0_ru_e, w4_ru_e = _prep_w(W_ru_e, L, 2 * U)
    w0_c_e, w4_c_e = _prep_w(W_c_e, L, U)
    w0_ru_d, w4_ru_d = _prep_w(W_ru_d, 1, 2 * U)
    w0_c_d, w4_c_d = _prep_w(W_c_d, 1, U)

    out = pl.pallas_call(
        _fwd_kernel,
        out_shape=jax.ShapeDtypeStruct((HORIZON, B * N), F32),
    )(x_all, support0.T.astype(BF16), support1.T.astype(BF16),
      w0_ru_e, w4_ru_e, b_ru_e.reshape(1, 2 * U),
      w0_c_e, w4_c_e, b_c_e.reshape(1, U),
      w0_ru_d, w4_ru_d, b_ru_d.reshape(1, 2 * U),
      w0_c_d, w4_c_d, b_c_d.reshape(1, U),
      W_proj.T, b_proj.reshape(1, 1))

    return out.reshape(HORIZON, B, N)


# speedup vs baseline: 1.1512x; 1.0016x over previous
import jax
import jax.numpy as jnp
from jax.experimental import pallas as pl

N = 207
B = 16
L = 8
HORIZON = 8
U = 64
NUM_MAT = 5
DIN = 80
F32 = jnp.float32
BF16 = jnp.bfloat16

_DNT = (((0,), (0,)), ((), ()))


def _fwd_kernel(x_all_ref, ss_ref, s0t_ref, s1t_ref,
                w0_ru_e_ref, w4_ru_e_ref, b_ru_e_ref,
                w0_c_e_ref, w4_c_e_ref, b_c_e_ref,
                w0_ru_d_ref, w4_ru_d_ref, b_ru_d_ref,
                w0_c_d_ref, w4_c_d_ref, b_c_d_ref,
                w_projt_ref, b_proj_ref,
                out_ref):
    ss = ss_ref[...]
    s0t = s0t_ref[...]
    s1t = s1t_ref[...]

    def dotT(a, w):
        return jax.lax.dot_general(a, w, _DNT, preferred_element_type=F32)

    def gconv(cat, w0, w4, b):
        g0 = jnp.dot(cat, w0, preferred_element_type=F32)
        cat3 = cat.reshape(B, N, DIN)
        big = jnp.concatenate(
            [dotT(cat3[i], ss).astype(BF16) for i in range(B)], axis=0)
        z1a = big[:, :N]
        z1b = big[:, 256:256 + N]
        z2a = jnp.dot(z1a, s0t, preferred_element_type=F32).astype(BF16)
        z2b = jnp.dot(z1b, s1t, preferred_element_type=F32).astype(BF16)
        gates = []
        for i in range(B):
            sl = slice(i * DIN, (i + 1) * DIN)
            zcat = jnp.concatenate([z1a[sl], z2a[sl], z1b[sl], z2b[sl]], axis=0)
            gates.append(dotT(zcat, w4))
        return g0 + jnp.concatenate(gates, axis=0) + b

    def cell(x16, h, w0_ru, w4_ru, b_ru, w0_c, w4_c, b_c):
        cat = jnp.concatenate([h.astype(BF16), x16], axis=1)
        ru = jax.nn.sigmoid(gconv(cat, w0_ru, w4_ru, b_ru))
        r = ru[:, :U]
        u = ru[:, U:]
        cat2 = jnp.concatenate([(r * h).astype(BF16), x16], axis=1)
        c = jnp.tanh(gconv(cat2, w0_c, w4_c, b_c))
        return u * h + (1.0 - u) * c

    w0_ru_e = w0_ru_e_ref[...]
    w4_ru_e = w4_ru_e_ref[...]
    b_ru_e = b_ru_e_ref[...]
    w0_c_e = w0_c_e_ref[...]
    w4_c_e = w4_c_e_ref[...]
    b_c_e = b_c_e_ref[...]

    def enc_body(t, h):
        x16 = x_all_ref[pl.ds(t, 1)].reshape(B * N, DIN - U)
        return cell(x16, h, w0_ru_e, w4_ru_e, b_ru_e, w0_c_e, w4_c_e, b_c_e)

    h = jax.lax.fori_loop(0, L, enc_body, jnp.zeros((B * N, U), F32))

    w0_ru_d = w0_ru_d_ref[...]
    w4_ru_d = w4_ru_d_ref[...]
    b_ru_d = b_ru_d_ref[...]
    w0_c_d = w0_c_d_ref[...]
    w4_c_d = w4_c_d_ref[...]
    b_c_d = b_c_d_ref[...]
    w_projt = w_projt_ref[...]
    b_proj = b_proj_ref[0, 0]

    def dec_body(t, h):
        projt = jnp.dot(w_projt, h.T, preferred_element_type=F32) + b_proj
        xin = jnp.where(t == 0, jnp.zeros_like(projt), projt).T
        x16 = jnp.pad(xin, ((0, 0), (0, DIN - U - 1))).astype(BF16)
        h2 = cell(x16, h, w0_ru_d, w4_ru_d, b_ru_d, w0_c_d, w4_c_d, b_c_d)
        proj2 = jnp.dot(w_projt, h2.T, preferred_element_type=F32) + b_proj
        out_ref[pl.ds(t, 1)] = proj2
        return h2

    jax.lax.fori_loop(0, HORIZON, dec_body, h)


def _prep_w(w, dx, dout):
    din = dx + U
    wm = w.reshape(din, NUM_MAT, dout)
    wm = jnp.concatenate(
        [wm[dx:], wm[:dx], jnp.zeros((DIN - din, NUM_MAT, dout), F32)],
        axis=0)
    w0 = wm[:, 0] - wm[:, 2] - wm[:, 4]
    w4 = jnp.concatenate(
        [wm[:, 1], 2.0 * wm[:, 2], wm[:, 3], 2.0 * wm[:, 4]], axis=0)
    return w0.astype(BF16), w4.astype(BF16)


def kernel(inputs, support0, support1, W_ru_e, b_ru_e, W_c_e, b_c_e,
           W_ru_d, b_ru_d, W_c_d, b_c_d, W_proj, b_proj):
    x_all = jnp.pad(inputs.reshape(L, B * N, L),
                    ((0, 0), (0, 0), (0, DIN - U - L))).astype(BF16)

    w0_ru_e, w4_ru_e = _prep_w(W_ru_e, L, 2 * U)
    w0_c_e, w4_c_e = _prep_w(W_c_e, L, U)
    w0_ru_d, w4_ru_d = _prep_w(W_ru_d, 1, 2 * U)
    w0_c_d, w4_c_d = _prep_w(W_c_d, 1, U)

    s0t = support0.T.astype(BF16)
    s1t = support1.T.astype(BF16)
    zpad = jnp.zeros((N, 256 - N), BF16)
    ss = jnp.concatenate([s0t, zpad, s1t, zpad], axis=1)

    out = pl.pallas_call(
        _fwd_kernel,
        out_shape=jax.ShapeDtypeStruct((HORIZON, B * N), F32),
    )(x_all, ss, s0t, s1t,
      w0_ru_e, w4_ru_e, b_ru_e.reshape(1, 2 * U),
      w0_c_e, w4_c_e, b_c_e.reshape(1, U),
      w0_ru_d, w4_ru_d, b_ru_d.reshape(1, 2 * U),
      w0_c_d, w4_c_d, b_c_d.reshape(1, U),
      W_proj.T, b_proj.reshape(1, 1))

    return out.reshape(HORIZON, B, N)
